# baseline (speedup 1.0000x reference)
import functools

import jax
import jax.numpy as jnp
from jax import lax
from jax.experimental import pallas as pl
from jax.experimental.pallas import tpu as pltpu
from jax.experimental.pallas import tpu_sc as plsc

NUM_CORES = 2
NUM_SUBCORES = 16
NUM_WORKERS = NUM_CORES * NUM_SUBCORES

VOCAB = 1000000
DIM = 32
B_TOTAL = 16384 * 50
B_PER_W = B_TOTAL // NUM_WORKERS
CHUNK = 1600
N_CHUNKS = B_PER_W // CHUNK

LANE = 128
SUB = 8
N_TCOLS = VOCAB // LANE
TAIL = VOCAB - N_TCOLS * LANE
TCOL_ITERS = (N_TCOLS + NUM_WORKERS - 1) // NUM_WORKERS


def _make_compact():
    mesh = plsc.VectorSubcoreMesh(core_axis_name="c", subcore_axis_name="s")

    @functools.partial(
        pl.kernel,
        out_type=jax.ShapeDtypeStruct((VOCAB * DIM,), jnp.float32),
        mesh=mesh,
        scratch_types=[
            pltpu.VMEM((DIM // SUB, SUB, LANE), jnp.float32),
            pltpu.VMEM((LANE * DIM,), jnp.float32),
            pltpu.VMEM((TAIL * DIM,), jnp.float32),
        ],
        compiler_params=pltpu.CompilerParams(
            use_tc_tiling_on_sc=True, needs_layout_passes=False),
    )
    def compact_kernel(wt_hbm, tail_hbm, w1_hbm, in_buf, out_buf, tail_buf):
        wid = lax.axis_index("s") * NUM_CORES + lax.axis_index("c")
        lane16 = lax.iota(jnp.int32, 16)
        scatter_idx = lane16 * DIM

        def do_tile_col(C):
            for R in range(DIM // SUB):
                pltpu.sync_copy(
                    wt_hbm.at[pl.ds(R * SUB, SUB), pl.ds(C * LANE, LANE)],
                    in_buf.at[R])
            for d in range(DIM):
                R, r = d // SUB, d % SUB
                for c0 in range(0, LANE, 16):
                    vals = in_buf[R, r, pl.ds(c0, 16)]
                    plsc.store_scatter(
                        out_buf, [scatter_idx + (c0 * DIM + d)], vals)
            pltpu.sync_copy(out_buf, w1_hbm.at[pl.ds(C * LANE * DIM, LANE * DIM)])

        def body(j, carry):
            C = j * NUM_WORKERS + wid

            @pl.when(C < N_TCOLS)
            def _():
                do_tile_col(C)

            return carry

        lax.fori_loop(0, TCOL_ITERS, body, 0)

        @pl.when(wid == 0)
        def _():
            pltpu.sync_copy(tail_hbm, tail_buf)
            pltpu.sync_copy(
                tail_buf,
                w1_hbm.at[pl.ds(N_TCOLS * LANE * DIM, TAIL * DIM)])

    return compact_kernel


def _make_gather():
    mesh = plsc.VectorSubcoreMesh(core_axis_name="c", subcore_axis_name="s")

    @functools.partial(
        pl.kernel,
        out_type=jax.ShapeDtypeStruct((B_TOTAL, DIM), jnp.float32),
        mesh=mesh,
        scratch_types=[
            pltpu.VMEM((CHUNK,), jnp.int32),
            pltpu.VMEM((CHUNK,), jnp.int32),
            pltpu.VMEM((CHUNK, DIM), jnp.float32),
            pltpu.VMEM((CHUNK, DIM), jnp.float32),
            pltpu.SemaphoreType.DMA,
            pltpu.SemaphoreType.DMA,
            pltpu.SemaphoreType.DMA,
            pltpu.SemaphoreType.DMA,
        ],
        compiler_params=pltpu.CompilerParams(use_tc_tiling_on_sc=False),
    )
    def gather_kernel(idx_hbm, table_hbm, out_hbm,
                      idx0, idx1, rows0, rows1, gsem0, gsem1, ssem0, ssem1):
        wid = lax.axis_index("s") * NUM_CORES + lax.axis_index("c")
        base = wid * B_PER_W
        idx_v = (idx0, idx1)
        rows_v = (rows0, rows1)
        gsem = (gsem0, gsem1)
        ssem = (ssem0, ssem1)
        gathers = [None] * N_CHUNKS
        stores = [None] * N_CHUNKS
        pltpu.sync_copy(idx_hbm.at[pl.ds(base, CHUNK)], idx_v[0])
        gathers[0] = pltpu.async_copy(table_hbm.at[idx_v[0]], rows_v[0], gsem[0])
        for i in range(N_CHUNKS):
            b = i & 1
            nb = 1 - b
            if i + 1 < N_CHUNKS:
                off = base + (i + 1) * CHUNK
                pltpu.sync_copy(idx_hbm.at[pl.ds(off, CHUNK)], idx_v[nb])
                if i >= 1:
                    stores[i - 1].wait()
                gathers[i + 1] = pltpu.async_copy(
                    table_hbm.at[idx_v[nb]], rows_v[nb], gsem[nb])
            gathers[i].wait()
            stores[i] = pltpu.async_copy(
                rows_v[b], out_hbm.at[pl.ds(base + i * CHUNK, CHUNK)], ssem[b])
        stores[N_CHUNKS - 2].wait()
        stores[N_CHUNKS - 1].wait()

    return gather_kernel


_compact = _make_compact()
_gather = _make_gather()


@jax.jit
def kernel(token_ids, weight):
    wt = weight.T
    tail = weight[N_TCOLS * LANE:].reshape(-1)
    w1 = _compact(wt, tail)
    flat_idx = token_ids.reshape(-1)
    out = _gather(flat_idx, w1.reshape(VOCAB, DIM))
    return out.reshape(token_ids.shape + (DIM,))


# speedup vs baseline: 1.0360x; 1.0360x over previous
import functools

import jax
import jax.numpy as jnp
from jax import lax
from jax.experimental import pallas as pl
from jax.experimental import layout as jex_layout
from jax.experimental.pallas import tpu as pltpu
from jax.experimental.pallas import tpu_sc as plsc

NUM_CORES = 2
NUM_SUBCORES = 16
NUM_WORKERS = NUM_CORES * NUM_SUBCORES

VOCAB = 1000000
DIM = 32
B_TOTAL = 16384 * 50
B_PER_W = B_TOTAL // NUM_WORKERS
CHUNK = 1600
N_CHUNKS = B_PER_W // CHUNK

SLAB = 512
N_SLABS = (VOCAB + SLAB - 1) // SLAB
W2_ROWS = N_SLABS * (SLAB // 4)
K_ROWS = W2_ROWS * 128 // DIM


def _transpose_body(wt_ref, out_ref):
    x = wt_ref[...]
    pieces = [x[:, u * 128:(u + 1) * 128].T for u in range(4)]
    out_ref[...] = jnp.concatenate(pieces, axis=1)


_compact = pl.pallas_call(
    _transpose_body,
    grid=(N_SLABS,),
    in_specs=[pl.BlockSpec((DIM, SLAB), lambda g: (0, g))],
    out_specs=pl.BlockSpec((SLAB // 4, 128), lambda g: (g, 0)),
    out_shape=jax.ShapeDtypeStruct((W2_ROWS, 128), jnp.float32),
)


def _make_gather():
    mesh = plsc.VectorSubcoreMesh(core_axis_name="c", subcore_axis_name="s")

    @functools.partial(
        pl.kernel,
        out_type=jax.ShapeDtypeStruct((B_TOTAL, DIM), jnp.float32),
        mesh=mesh,
        scratch_types=[
            pltpu.VMEM((CHUNK,), jnp.int32),
            pltpu.VMEM((CHUNK,), jnp.int32),
            pltpu.VMEM((CHUNK, DIM), jnp.float32),
            pltpu.VMEM((CHUNK, DIM), jnp.float32),
            pltpu.SemaphoreType.DMA,
            pltpu.SemaphoreType.DMA,
            pltpu.SemaphoreType.DMA,
            pltpu.SemaphoreType.DMA,
        ],
        compiler_params=pltpu.CompilerParams(use_tc_tiling_on_sc=False),
    )
    def gather_kernel(idx_hbm, table_hbm, out_hbm,
                      idx0, idx1, rows0, rows1, gsem0, gsem1, ssem0, ssem1):
        wid = lax.axis_index("s") * NUM_CORES + lax.axis_index("c")
        base = wid * B_PER_W
        idx_v = (idx0, idx1)
        rows_v = (rows0, rows1)
        gsem = (gsem0, gsem1)
        ssem = (ssem0, ssem1)
        gathers = [None] * N_CHUNKS
        stores = [None] * N_CHUNKS
        pltpu.sync_copy(idx_hbm.at[pl.ds(base, CHUNK)], idx_v[0])
        gathers[0] = pltpu.async_copy(table_hbm.at[idx_v[0]], rows_v[0], gsem[0])
        for i in range(N_CHUNKS):
            b = i & 1
            nb = 1 - b
            if i + 1 < N_CHUNKS:
                off = base + (i + 1) * CHUNK
                pltpu.sync_copy(idx_hbm.at[pl.ds(off, CHUNK)], idx_v[nb])
                if i >= 1:
                    stores[i - 1].wait()
                gathers[i + 1] = pltpu.async_copy(
                    table_hbm.at[idx_v[nb]], rows_v[nb], gsem[nb])
            gathers[i].wait()
            stores[i] = pltpu.async_copy(
                rows_v[b], out_hbm.at[pl.ds(base + i * CHUNK, CHUNK)], ssem[b])
        stores[N_CHUNKS - 2].wait()
        stores[N_CHUNKS - 1].wait()

    return gather_kernel


_gather = _make_gather()


def _kernel_impl(token_ids, weight):
    w2 = _compact(weight.T)
    v = token_ids.reshape(-1)
    k = (v >> 9) * 512 + (v & 127) * 4 + ((v >> 7) & 3)
    out = _gather(k, w2.reshape(K_ROWS, DIM))
    return out.reshape(token_ids.shape + (DIM,))


kernel = jax.jit(_kernel_impl)


# speedup vs baseline: 1.2150x; 1.1727x over previous
import functools

import jax
import jax.numpy as jnp
from jax import lax
from jax.experimental import pallas as pl
from jax.experimental import layout as jex_layout
from jax.experimental.pallas import tpu as pltpu
from jax.experimental.pallas import tpu_sc as plsc

NUM_CORES = 2
NUM_SUBCORES = 16
NUM_WORKERS = NUM_CORES * NUM_SUBCORES

VOCAB = 1000000
DIM = 32
B_TOTAL = 16384 * 50
B_PER_W = B_TOTAL // NUM_WORKERS
CHUNK = 1600
N_CHUNKS = B_PER_W // CHUNK

SLAB = 512
N_SLABS = (VOCAB + SLAB - 1) // SLAB
W2_ROWS = N_SLABS * (SLAB // 4)
K_ROWS = W2_ROWS * 128 // DIM


def _transpose_body(wt_ref, out_ref):
    x = wt_ref[...]
    pieces = [x[:, u * 128:(u + 1) * 128].T for u in range(4)]
    out_ref[...] = jnp.concatenate(pieces, axis=1)


_compact = pl.pallas_call(
    _transpose_body,
    grid=(N_SLABS,),
    in_specs=[pl.BlockSpec((DIM, SLAB), lambda g: (0, g))],
    out_specs=pl.BlockSpec((SLAB // 4, 128), lambda g: (g, 0)),
    out_shape=jax.ShapeDtypeStruct((W2_ROWS, 128), jnp.float32),
)


T_COUNT = 50
B_COUNT = 16384
BCHUNK = 256
UNITS_PER_T = B_COUNT // BCHUNK
UNITS_PER_W = B_TOTAL // BCHUNK // NUM_WORKERS
OUT_FLOATS = B_TOTAL * DIM


def _make_gather():
    mesh = plsc.VectorSubcoreMesh(core_axis_name="c", subcore_axis_name="s")

    @functools.partial(
        pl.kernel,
        out_type=jax.ShapeDtypeStruct((OUT_FLOATS,), jnp.float32),
        mesh=mesh,
        scratch_types=[
            pltpu.VMEM((B_PER_W,), jnp.int32),
            pltpu.VMEM((BCHUNK, DIM), jnp.float32),
            pltpu.VMEM((BCHUNK, DIM), jnp.float32),
            pltpu.VMEM((BCHUNK * DIM,), jnp.float32),
            pltpu.VMEM((BCHUNK * DIM,), jnp.float32),
            pltpu.SemaphoreType.DMA,
            pltpu.SemaphoreType.DMA,
            pltpu.SemaphoreType.DMA,
            pltpu.SemaphoreType.DMA,
        ],
        compiler_params=pltpu.CompilerParams(
            use_tc_tiling_on_sc=False, needs_layout_passes=False),
    )
    def gather_kernel(idx_hbm, table_hbm, out_hbm, idx_all,
                      rowsA, rowsB, tbufA, tbufB,
                      gsemA, gsemB, ssemA, ssemB):
        wid = lax.axis_index("s") * NUM_CORES + lax.axis_index("c")
        lane16 = lax.iota(jnp.int32, 16)
        pltpu.sync_copy(idx_hbm.at[pl.ds(wid * B_PER_W, B_PER_W)], idx_all)

        def start_gather(u, rows, gsem):
            return pltpu.async_copy(
                table_hbm.at[idx_all.at[pl.ds(u * BCHUNK, BCHUNK)]],
                rows, gsem)

        def drain_gather(rows, gsem):
            pltpu.make_async_copy(
                table_hbm.at[pl.ds(0, BCHUNK)], rows, gsem).wait()

        def drain_stores(tbuf, ssem):
            pltpu.make_async_copy(
                out_hbm.at[pl.ds(0, BCHUNK * DIM)], tbuf, ssem).wait()

        def process(u, rows, tbuf, ssem):
            for d in range(DIM):
                dvec = jnp.full((16,), d, jnp.int32)
                for c0 in range(0, BCHUNK, 16):
                    vals = plsc.load_gather(rows, [lane16 + c0, dvec])
                    tbuf[pl.ds(d * BCHUNK + c0, 16)] = vals
            U = wid * UNITS_PER_W + u
            t = U // UNITS_PER_T
            bb = U % UNITS_PER_T
            base = t * (DIM * B_COUNT) + bb * BCHUNK
            for d in range(DIM):
                pltpu.async_copy(
                    tbuf.at[pl.ds(d * BCHUNK, BCHUNK)],
                    out_hbm.at[pl.ds(base + d * B_COUNT, BCHUNK)], ssem)

        start_gather(0, rowsA, gsemA)

        def body(k, carry):
            uA = 2 * k
            uB = 2 * k + 1
            start_gather(uB, rowsB, gsemB)
            drain_gather(rowsA, gsemA)

            @pl.when(k > 0)
            def _():
                drain_stores(tbufA, ssemA)

            process(uA, rowsA, tbufA, ssemA)

            @pl.when(k < UNITS_PER_W // 2 - 1)
            def _():
                start_gather(uA + 2, rowsA, gsemA)

            drain_gather(rowsB, gsemB)

            @pl.when(k > 0)
            def _():
                drain_stores(tbufB, ssemB)

            process(uB, rowsB, tbufB, ssemB)
            return carry

        lax.fori_loop(0, UNITS_PER_W // 2, body, 0)
        drain_stores(tbufA, ssemA)
        drain_stores(tbufB, ssemB)

    return gather_kernel


_gather = _make_gather()


def _kernel_impl(token_ids, weight):
    w2 = _compact(weight.T)
    v = token_ids
    k = (v >> 9) * 512 + (v & 127) * 4 + ((v >> 7) & 3)
    kflat = k.T.reshape(-1)
    out1 = _gather(kflat, w2.reshape(K_ROWS, DIM))
    return jnp.transpose(out1.reshape(T_COUNT, DIM, B_COUNT), (2, 0, 1))


kernel = jax.jit(_kernel_impl)


# speedup vs baseline: 1.2944x; 1.0654x over previous
import functools

import jax
import jax.numpy as jnp
from jax import lax
from jax.experimental import pallas as pl
from jax.experimental import layout as jex_layout
from jax.experimental.pallas import tpu as pltpu
from jax.experimental.pallas import tpu_sc as plsc

NUM_CORES = 2
NUM_SUBCORES = 16
NUM_WORKERS = NUM_CORES * NUM_SUBCORES

VOCAB = 1000000
DIM = 32
B_TOTAL = 16384 * 50
B_PER_W = B_TOTAL // NUM_WORKERS
CHUNK = 1600
N_CHUNKS = B_PER_W // CHUNK

SLAB = 512
N_SLABS = (VOCAB + SLAB - 1) // SLAB
W2_ROWS = N_SLABS * (SLAB // 4)
K_ROWS = W2_ROWS * 128 // DIM


def _transpose_body(wt_ref, out_ref):
    x = wt_ref[...]
    eye = jnp.eye(DIM, dtype=jnp.float32)
    for u in range(4):
        piece = lax.dot_general(
            x[:, u * 128:(u + 1) * 128], eye,
            (((0,), (0,)), ((), ())),
            preferred_element_type=jnp.float32)
        out_ref[:, u * DIM:(u + 1) * DIM] = piece


_compact = pl.pallas_call(
    _transpose_body,
    grid=(N_SLABS,),
    in_specs=[pl.BlockSpec((DIM, SLAB), lambda g: (0, g))],
    out_specs=pl.BlockSpec((SLAB // 4, 128), lambda g: (g, 0)),
    out_shape=jax.ShapeDtypeStruct((W2_ROWS, 128), jnp.float32),
    compiler_params=pltpu.CompilerParams(fuse_transposed_lhs_in_matmul=True),
)


T_COUNT = 50
B_COUNT = 16384
BCHUNK = 256
UNITS_PER_T = B_COUNT // BCHUNK
UNITS_PER_W = B_TOTAL // BCHUNK // NUM_WORKERS
OUT_FLOATS = B_TOTAL * DIM


def _make_gather():
    mesh = plsc.VectorSubcoreMesh(core_axis_name="c", subcore_axis_name="s")

    @functools.partial(
        pl.kernel,
        out_type=jax.ShapeDtypeStruct((OUT_FLOATS,), jnp.float32),
        mesh=mesh,
        scratch_types=[
            pltpu.VMEM((B_PER_W,), jnp.int32),
            pltpu.VMEM((BCHUNK, DIM), jnp.float32),
            pltpu.VMEM((BCHUNK, DIM), jnp.float32),
            pltpu.VMEM((BCHUNK * DIM,), jnp.float32),
            pltpu.VMEM((BCHUNK * DIM,), jnp.float32),
            pltpu.SemaphoreType.DMA,
            pltpu.SemaphoreType.DMA,
            pltpu.SemaphoreType.DMA,
            pltpu.SemaphoreType.DMA,
        ],
        compiler_params=pltpu.CompilerParams(
            use_tc_tiling_on_sc=False, needs_layout_passes=False),
    )
    def gather_kernel(idx_hbm, table_hbm, out_hbm, idx_all,
                      rowsA, rowsB, tbufA, tbufB,
                      gsemA, gsemB, ssemA, ssemB):
        wid = lax.axis_index("s") * NUM_CORES + lax.axis_index("c")
        lane16 = lax.iota(jnp.int32, 16)
        pltpu.sync_copy(idx_hbm.at[pl.ds(wid * B_PER_W, B_PER_W)], idx_all)

        def start_gather(u, rows, gsem):
            return pltpu.async_copy(
                table_hbm.at[idx_all.at[pl.ds(u * BCHUNK, BCHUNK)]],
                rows, gsem)

        def drain_gather(rows, gsem):
            pltpu.make_async_copy(
                table_hbm.at[pl.ds(0, BCHUNK)], rows, gsem).wait()

        def drain_stores(tbuf, ssem):
            pltpu.make_async_copy(
                out_hbm.at[pl.ds(0, BCHUNK * DIM)], tbuf, ssem).wait()

        def process(u, rows, tbuf, ssem):
            scat0 = lane16 * BCHUNK
            scat1 = (lane16 + 16) * BCHUNK
            for c in range(BCHUNK):
                plsc.store_scatter(tbuf, [scat0 + c], rows[c, pl.ds(0, 16)])
                plsc.store_scatter(tbuf, [scat1 + c], rows[c, pl.ds(16, 16)])
            U = wid * UNITS_PER_W + u
            t = U // UNITS_PER_T
            bb = U % UNITS_PER_T
            base = t * (DIM * B_COUNT) + bb * BCHUNK
            for d in range(DIM):
                pltpu.async_copy(
                    tbuf.at[pl.ds(d * BCHUNK, BCHUNK)],
                    out_hbm.at[pl.ds(base + d * B_COUNT, BCHUNK)], ssem)

        start_gather(0, rowsA, gsemA)

        def body(k, carry):
            uA = 2 * k
            uB = 2 * k + 1
            start_gather(uB, rowsB, gsemB)
            drain_gather(rowsA, gsemA)

            @pl.when(k > 0)
            def _():
                drain_stores(tbufA, ssemA)

            process(uA, rowsA, tbufA, ssemA)

            @pl.when(k < UNITS_PER_W // 2 - 1)
            def _():
                start_gather(uA + 2, rowsA, gsemA)

            drain_gather(rowsB, gsemB)

            @pl.when(k > 0)
            def _():
                drain_stores(tbufB, ssemB)

            process(uB, rowsB, tbufB, ssemB)
            return carry

        lax.fori_loop(0, UNITS_PER_W // 2, body, 0)
        drain_stores(tbufA, ssemA)
        drain_stores(tbufB, ssemB)

    return gather_kernel


_gather = _make_gather()


def _kernel_impl(token_ids, weight):
    w2 = _compact(weight.T)
    v = token_ids
    k = (v >> 9) * 512 + (v & 127) * 4 + ((v >> 7) & 3)
    kflat = k.T.reshape(-1)
    out1 = _gather(kflat, w2.reshape(K_ROWS, DIM))
    return jnp.transpose(out1.reshape(T_COUNT, DIM, B_COUNT), (2, 0, 1))


kernel = jax.jit(_kernel_impl)


# speedup vs baseline: 2.5908x; 2.0015x over previous
import functools

import jax
import jax.numpy as jnp
from jax import lax
from jax.experimental import pallas as pl
from jax.experimental import layout as jex_layout
from jax.experimental.pallas import tpu as pltpu
from jax.experimental.pallas import tpu_sc as plsc

NUM_CORES = 2
NUM_SUBCORES = 16
NUM_WORKERS = NUM_CORES * NUM_SUBCORES

VOCAB = 1000000
DIM = 32
B_TOTAL = 16384 * 50
B_PER_W = B_TOTAL // NUM_WORKERS
CHUNK = 1600
N_CHUNKS = B_PER_W // CHUNK

SLAB = 512
SUPER = 8
N_SUPER = (VOCAB + SLAB * SUPER - 1) // (SLAB * SUPER)
W2_ROWS = N_SUPER * SUPER * (SLAB // 4)
K_ROWS = W2_ROWS * 128 // DIM


def _transpose_body(wt_ref, out_ref):
    x = wt_ref[...]
    eye = jnp.eye(DIM, dtype=jnp.float32)
    for s in range(SUPER):
        for u in range(4):
            c0 = s * SLAB + u * 128
            piece = lax.dot_general(
                x[:, c0:c0 + 128], eye,
                (((0,), (0,)), ((), ())),
                preferred_element_type=jnp.float32)
            out_ref[s * 128:(s + 1) * 128, u * DIM:(u + 1) * DIM] = piece


_compact = pl.pallas_call(
    _transpose_body,
    grid=(N_SUPER,),
    in_specs=[pl.BlockSpec((DIM, SLAB * SUPER), lambda g: (0, g))],
    out_specs=pl.BlockSpec((SUPER * SLAB // 4, 128), lambda g: (g, 0)),
    out_shape=jax.ShapeDtypeStruct((W2_ROWS, 128), jnp.float32),
    compiler_params=pltpu.CompilerParams(fuse_transposed_lhs_in_matmul=True),
)


T_COUNT = 50
B_COUNT = 16384
BCHUNK = 256
UNITS_PER_T = B_COUNT // BCHUNK
UNITS_PER_W = B_TOTAL // BCHUNK // NUM_WORKERS
OUT_FLOATS = B_TOTAL * DIM


def _make_gather():
    mesh = plsc.VectorSubcoreMesh(core_axis_name="c", subcore_axis_name="s")

    @functools.partial(
        pl.kernel,
        out_type=jax.ShapeDtypeStruct((OUT_FLOATS,), jnp.float32),
        mesh=mesh,
        scratch_types=[
            pltpu.VMEM((B_PER_W,), jnp.int32),
            pltpu.VMEM((BCHUNK, DIM), jnp.float32),
            pltpu.VMEM((BCHUNK, DIM), jnp.float32),
            pltpu.VMEM((BCHUNK * DIM // 2,), jnp.float32),
            pltpu.VMEM((BCHUNK * DIM // 2,), jnp.float32),
            pltpu.VMEM((BCHUNK * DIM // 2,), jnp.float32),
            pltpu.VMEM((BCHUNK * DIM // 2,), jnp.float32),
            pltpu.SemaphoreType.DMA,
            pltpu.SemaphoreType.DMA,
            pltpu.SemaphoreType.DMA,
            pltpu.SemaphoreType.DMA,
        ],
        compiler_params=pltpu.CompilerParams(
            use_tc_tiling_on_sc=False, needs_layout_passes=False),
    )
    def gather_kernel(idx_hbm, table_hbm, out_hbm, idx_all,
                      rowsA, rowsB, tbufA_lo, tbufA_hi, tbufB_lo, tbufB_hi,
                      gsemA, gsemB, ssemA, ssemB):
        wid = lax.axis_index("s") * NUM_CORES + lax.axis_index("c")
        lane16 = lax.iota(jnp.int32, 16)
        pltpu.sync_copy(idx_hbm.at[pl.ds(wid * B_PER_W, B_PER_W)], idx_all)

        def start_gather(u, rows, gsem):
            return pltpu.async_copy(
                table_hbm.at[idx_all.at[pl.ds(u * BCHUNK, BCHUNK)]],
                rows, gsem)

        def drain_gather(rows, gsem):
            pltpu.make_async_copy(
                table_hbm.at[pl.ds(0, BCHUNK)], rows, gsem).wait()

        def drain_stores(tbuf_lo, tbuf_hi, ssem):
            pltpu.make_async_copy(
                out_hbm.at[pl.ds(0, BCHUNK * DIM // 2)], tbuf_lo, ssem).wait()
            pltpu.make_async_copy(
                out_hbm.at[pl.ds(0, BCHUNK * DIM // 2)], tbuf_hi, ssem).wait()

        def process(u, rows, tbuf_lo, tbuf_hi, ssem):
            scat = lane16 * BCHUNK
            for c in range(BCHUNK):
                plsc.store_scatter(tbuf_lo, [scat + c], rows[c, pl.ds(0, 16)])
                plsc.store_scatter(tbuf_hi, [scat + c], rows[c, pl.ds(16, 16)])
            U = wid * UNITS_PER_W + u
            t = U // UNITS_PER_T
            bb = U % UNITS_PER_T
            base = t * (DIM * B_COUNT) + bb * BCHUNK
            for d in range(DIM):
                tb = tbuf_lo if d < 16 else tbuf_hi
                pltpu.async_copy(
                    tb.at[pl.ds((d % 16) * BCHUNK, BCHUNK)],
                    out_hbm.at[pl.ds(base + d * B_COUNT, BCHUNK)], ssem)

        start_gather(0, rowsA, gsemA)

        def body(k, carry):
            uA = 2 * k
            uB = 2 * k + 1
            start_gather(uB, rowsB, gsemB)
            drain_gather(rowsA, gsemA)

            @pl.when(k > 0)
            def _():
                drain_stores(tbufA_lo, tbufA_hi, ssemA)

            process(uA, rowsA, tbufA_lo, tbufA_hi, ssemA)

            @pl.when(k < UNITS_PER_W // 2 - 1)
            def _():
                start_gather(uA + 2, rowsA, gsemA)

            drain_gather(rowsB, gsemB)

            @pl.when(k > 0)
            def _():
                drain_stores(tbufB_lo, tbufB_hi, ssemB)

            process(uB, rowsB, tbufB_lo, tbufB_hi, ssemB)
            return carry

        lax.fori_loop(0, UNITS_PER_W // 2, body, 0)
        drain_stores(tbufA_lo, tbufA_hi, ssemA)
        drain_stores(tbufB_lo, tbufB_hi, ssemB)

    return gather_kernel


_gather = _make_gather()


def _kernel_impl(token_ids, weight):
    w2 = _compact(weight.T)
    v = token_ids
    k = (v >> 9) * 512 + (v & 127) * 4 + ((v >> 7) & 3)
    kflat = k.T.reshape(-1)
    out1 = _gather(kflat, w2.reshape(K_ROWS, DIM))
    return jnp.transpose(out1.reshape(T_COUNT, DIM, B_COUNT), (2, 0, 1))


kernel = jax.jit(_kernel_impl)


# speedup vs baseline: 2.5952x; 1.0017x over previous
import functools

import jax
import jax.numpy as jnp
from jax import lax
from jax.experimental import pallas as pl
from jax.experimental.pallas import tpu as pltpu
from jax.experimental.pallas import tpu_sc as plsc

NUM_CORES = 2
NUM_SUBCORES = 16
NUM_WORKERS = NUM_CORES * NUM_SUBCORES

VOCAB = 1000000
DIM = 32
B_TOTAL = 16384 * 50
B_PER_W = B_TOTAL // NUM_WORKERS
SLAB = 512
SUPER = 8
N_SUPER = (VOCAB + SLAB * SUPER - 1) // (SLAB * SUPER)
W2_ROWS = N_SUPER * SUPER * (SLAB // 4)
K_ROWS = W2_ROWS * 128 // DIM


def _transpose_body(wt_ref, out_ref):
    x = wt_ref[...]
    eye = jnp.eye(DIM, dtype=jnp.float32)
    for s in range(SUPER):
        for u in range(4):
            c0 = s * SLAB + u * 128
            piece = lax.dot_general(
                x[:, c0:c0 + 128], eye,
                (((0,), (0,)), ((), ())),
                preferred_element_type=jnp.float32)
            out_ref[s * 128:(s + 1) * 128, u * DIM:(u + 1) * DIM] = piece


_compact = pl.pallas_call(
    _transpose_body,
    grid=(N_SUPER,),
    in_specs=[pl.BlockSpec((DIM, SLAB * SUPER), lambda g: (0, g))],
    out_specs=pl.BlockSpec((SUPER * SLAB // 4, 128), lambda g: (g, 0)),
    out_shape=jax.ShapeDtypeStruct((W2_ROWS, 128), jnp.float32),
    compiler_params=pltpu.CompilerParams(fuse_transposed_lhs_in_matmul=True),
)


T_COUNT = 50
B_COUNT = 16384
BCHUNK = 256
UNITS_PER_T = B_COUNT // BCHUNK
UNITS_PER_W = B_TOTAL // BCHUNK // NUM_WORKERS
OUT_FLOATS = B_TOTAL * DIM


def _make_gather():
    mesh = plsc.VectorSubcoreMesh(core_axis_name="c", subcore_axis_name="s")

    @functools.partial(
        pl.kernel,
        out_type=jax.ShapeDtypeStruct((OUT_FLOATS,), jnp.float32),
        mesh=mesh,
        scratch_types=[
            pltpu.VMEM((B_PER_W,), jnp.int32),
            pltpu.VMEM((BCHUNK, DIM), jnp.float32),
            pltpu.VMEM((BCHUNK, DIM), jnp.float32),
            pltpu.VMEM((BCHUNK * DIM // 2,), jnp.float32),
            pltpu.VMEM((BCHUNK * DIM // 2,), jnp.float32),
            pltpu.VMEM((BCHUNK * DIM // 2,), jnp.float32),
            pltpu.VMEM((BCHUNK * DIM // 2,), jnp.float32),
            pltpu.SemaphoreType.DMA,
            pltpu.SemaphoreType.DMA,
            pltpu.SemaphoreType.DMA,
            pltpu.SemaphoreType.DMA,
        ],
        compiler_params=pltpu.CompilerParams(
            use_tc_tiling_on_sc=False, needs_layout_passes=False),
    )
    def gather_kernel(idx_hbm, table_hbm, out_hbm, idx_all,
                      rowsA, rowsB, tbufA_lo, tbufA_hi, tbufB_lo, tbufB_hi,
                      gsemA, gsemB, ssemA, ssemB):
        wid = lax.axis_index("s") * NUM_CORES + lax.axis_index("c")
        lane16 = lax.iota(jnp.int32, 16)
        pltpu.sync_copy(idx_hbm.at[pl.ds(wid * B_PER_W, B_PER_W)], idx_all)

        def start_gather(u, rows, gsem):
            return pltpu.async_copy(
                table_hbm.at[idx_all.at[pl.ds(u * BCHUNK, BCHUNK)]],
                rows, gsem)

        def drain_gather(rows, gsem):
            pltpu.make_async_copy(
                table_hbm.at[pl.ds(0, BCHUNK)], rows, gsem).wait()

        def drain_stores(tbuf_lo, tbuf_hi, ssem):
            pltpu.make_async_copy(
                out_hbm.at[pl.ds(0, BCHUNK * DIM // 2)], tbuf_lo, ssem).wait()
            pltpu.make_async_copy(
                out_hbm.at[pl.ds(0, BCHUNK * DIM // 2)], tbuf_hi, ssem).wait()

        def process(u, rows, tbuf_lo, tbuf_hi, ssem):
            scat = lane16 * BCHUNK
            for c in range(BCHUNK):
                plsc.store_scatter(tbuf_lo, [scat + c], rows[c, pl.ds(0, 16)])
                plsc.store_scatter(tbuf_hi, [scat + c], rows[c, pl.ds(16, 16)])
            U = wid * UNITS_PER_W + u
            t = U // UNITS_PER_T
            bb = U % UNITS_PER_T
            base = t * (DIM * B_COUNT) + bb * BCHUNK
            for d in range(DIM):
                tb = tbuf_lo if d < 16 else tbuf_hi
                pltpu.async_copy(
                    tb.at[pl.ds((d % 16) * BCHUNK, BCHUNK)],
                    out_hbm.at[pl.ds(base + d * B_COUNT, BCHUNK)], ssem)

        start_gather(0, rowsA, gsemA)

        def body(k, carry):
            uA = 2 * k
            uB = 2 * k + 1
            start_gather(uB, rowsB, gsemB)
            drain_gather(rowsA, gsemA)

            @pl.when(k > 0)
            def _():
                drain_stores(tbufA_lo, tbufA_hi, ssemA)

            process(uA, rowsA, tbufA_lo, tbufA_hi, ssemA)

            @pl.when(k < UNITS_PER_W // 2 - 1)
            def _():
                start_gather(uA + 2, rowsA, gsemA)

            drain_gather(rowsB, gsemB)

            @pl.when(k > 0)
            def _():
                drain_stores(tbufB_lo, tbufB_hi, ssemB)

            process(uB, rowsB, tbufB_lo, tbufB_hi, ssemB)
            return carry

        lax.fori_loop(0, UNITS_PER_W // 2, body, 0)
        drain_stores(tbufA_lo, tbufA_hi, ssemA)
        drain_stores(tbufB_lo, tbufB_hi, ssemB)

    return gather_kernel


_gather = _make_gather()


def _kernel_impl(token_ids, weight):
    w2 = _compact(weight.T)
    v = token_ids
    k = (v >> 9) * 512 + (v & 127) * 4 + ((v >> 7) & 3)
    kflat = k.T.reshape(-1)
    out1 = _gather(kflat, w2.reshape(K_ROWS, DIM))
    return jnp.transpose(out1.reshape(T_COUNT, DIM, B_COUNT), (2, 0, 1))


kernel = jax.jit(_kernel_impl)
